# baseline (speedup 1.0000x reference)
import functools

import numpy as np
import jax
import jax.numpy as jnp
from jax import lax
from jax.experimental import pallas as pl
from jax.experimental.pallas import tpu as pltpu
from jax.experimental.pallas import tpu_sc as plsc

_D = 768
_MAX_LEN = 2048
_BATCH = 4
_SCALE = float(np.sqrt(np.float32(_D)))


def _positional_encoding() -> np.ndarray:
    pos = np.arange(_MAX_LEN)[:, np.newaxis].astype(np.float64)
    i = np.arange(_D)[np.newaxis, :].astype(np.float64)
    angle_rates = 1.0 / np.power(10000.0, 2.0 * (i // 2) / np.float32(_D))
    angle_rads = pos * angle_rates
    angle_rads[:, 0::2] = np.sin(angle_rads[:, 0::2])
    angle_rads[:, 1::2] = np.cos(angle_rads[:, 1::2])
    return angle_rads.astype(np.float32)


_POS_ENC = _positional_encoding()

_INFO = plsc.get_sparse_core_info()
_NC = _INFO.num_cores
_NS = _INFO.num_subcores
_NW = _NC * _NS
_T_PER_W = _MAX_LEN // _NW
_TCH = 8
_NCH = _T_PER_W // _TCH
_NRING = 4
_PREF = 2
_LANES = 16
_COLS = _D // _LANES

_mesh = plsc.VectorSubcoreMesh(core_axis_name="c", subcore_axis_name="s")


@functools.partial(
    pl.kernel,
    mesh=_mesh,
    out_type=jax.ShapeDtypeStruct((_BATCH, _MAX_LEN, _D), jnp.float32),
    scratch_types=[
        pltpu.VMEM((_BATCH, _T_PER_W), jnp.int32),
        *[pltpu.VMEM((_TCH, _D), jnp.float32) for _ in range(_NRING)],
        *[pltpu.VMEM((_TCH, _D), jnp.float32)
          for _ in range(_NRING * _BATCH)],
        *[pltpu.SemaphoreType.DMA for _ in range(2 * _NRING)],
        *[pltpu.SemaphoreType.DMA for _ in range(_NRING * _BATCH)],
        pltpu.SemaphoreType.DMA,
    ],
)
def _embed(x_hbm, table_hbm, pos_hbm, out_hbm, idx_v, *refs):
    pos_v = refs[:_NRING]
    rows_v = [refs[_NRING + g * _BATCH:_NRING + (g + 1) * _BATCH]
              for g in range(_NRING)]
    psem = refs[_NRING * (1 + _BATCH):_NRING * (2 + _BATCH)]
    wsem = [refs[_NRING * (2 + _BATCH) + g * _BATCH:]
            [:_BATCH] for g in range(_NRING)]
    gsem = refs[_NRING * (2 + _BATCH) + _NRING * _BATCH:][:_NRING]
    isem = refs[-1]

    wid = lax.axis_index("s") * _NC + lax.axis_index("c")
    t_base = wid * _T_PER_W

    idx_cps = [
        pltpu.async_copy(x_hbm.at[b, pl.ds(t_base, _T_PER_W)],
                         idx_v.at[b], isem)
        for b in range(_BATCH)
    ]
    for cp in idx_cps:
        cp.wait()

    def fire_chunk(ct):
        g = ct % _NRING
        t0 = t_base + ct * _TCH
        row_cps = []
        for b in range(_BATCH):
            idx = idx_v.at[b, pl.ds(ct * _TCH, _TCH)]
            row_cps.append(
                pltpu.async_copy(table_hbm.at[idx], rows_v[g][b], gsem[g]))
        pos_cp = pltpu.async_copy(pos_hbm.at[pl.ds(t0, _TCH)], pos_v[g],
                                  psem[g])
        return pos_cp, row_cps

    chunks = {k: fire_chunk(k) for k in range(_PREF)}
    writes = {}
    for ct in range(_NCH):
        g = ct % _NRING
        if ct + _PREF < _NCH:
            if ct + _PREF - _NRING in writes:
                for w in writes[ct + _PREF - _NRING]:
                    w.wait()
            chunks[ct + _PREF] = fire_chunk(ct + _PREF)
        pos_cp, row_cps = chunks[ct]
        pos_cp.wait()
        for cp in row_cps:
            cp.wait()

        pv = pos_v[g]
        bufs = rows_v[g]

        @plsc.parallel_loop(0, _TCH * _COLS, unroll=4)
        def _slice(i):
            r = i // _COLS
            col = i - r * _COLS
            sl = pl.ds(col * _LANES, _LANES)
            p = pv[r, sl]
            for b in range(_BATCH):
                rb = bufs[b]
                rb[r, sl] = rb[r, sl] * _SCALE + p

        t0 = t_base + ct * _TCH
        writes[ct] = [
            pltpu.async_copy(bufs[b], out_hbm.at[b, pl.ds(t0, _TCH)],
                             wsem[g][b])
            for b in range(_BATCH)
        ]
    for ct in range(_NCH - _NRING, _NCH):
        for w in writes[ct]:
            w.wait()


def kernel(x, table):
    return _embed(x.astype(jnp.int32), table, jnp.asarray(_POS_ENC))


# speedup vs baseline: 1.0042x; 1.0042x over previous
import functools

import numpy as np
import jax
import jax.numpy as jnp
from jax import lax
from jax.experimental import pallas as pl
from jax.experimental.pallas import tpu as pltpu
from jax.experimental.pallas import tpu_sc as plsc

_D = 768
_MAX_LEN = 2048
_BATCH = 4
_SCALE = float(np.sqrt(np.float32(_D)))


def _positional_encoding() -> np.ndarray:
    pos = np.arange(_MAX_LEN)[:, np.newaxis].astype(np.float64)
    i = np.arange(_D)[np.newaxis, :].astype(np.float64)
    angle_rates = 1.0 / np.power(10000.0, 2.0 * (i // 2) / np.float32(_D))
    angle_rads = pos * angle_rates
    angle_rads[:, 0::2] = np.sin(angle_rads[:, 0::2])
    angle_rads[:, 1::2] = np.cos(angle_rads[:, 1::2])
    return angle_rads.astype(np.float32)


_POS_ENC = _positional_encoding()

_INFO = plsc.get_sparse_core_info()
_NC = _INFO.num_cores
_NS = _INFO.num_subcores
_NW = _NC * _NS
_T_PER_W = _MAX_LEN // _NW
_TCH = 8
_NCH = _T_PER_W // _TCH
_NRING = 4
_PREF = 2
_LANES = 16
_COLS = _D // _LANES

_mesh = plsc.VectorSubcoreMesh(core_axis_name="c", subcore_axis_name="s")


@functools.partial(
    pl.kernel,
    mesh=_mesh,
    out_type=jax.ShapeDtypeStruct((_BATCH, _MAX_LEN, _D), jnp.float32),
    scratch_types=[
        pltpu.VMEM((_BATCH, _T_PER_W), jnp.int32),
        *[pltpu.VMEM((_TCH, _D), jnp.float32) for _ in range(_NRING)],
        *[pltpu.VMEM((_TCH, _D), jnp.float32)
          for _ in range(_NRING * _BATCH)],
        *[pltpu.SemaphoreType.DMA for _ in range(2 * _NRING)],
        *[pltpu.SemaphoreType.DMA for _ in range(_NRING * _BATCH)],
        pltpu.SemaphoreType.DMA,
    ],
)
def _embed(x_hbm, table_hbm, pos_hbm, out_hbm, idx_v, *refs):
    pos_v = refs[:_NRING]
    rows_v = [refs[_NRING + g * _BATCH:_NRING + (g + 1) * _BATCH]
              for g in range(_NRING)]
    psem = refs[_NRING * (1 + _BATCH):_NRING * (2 + _BATCH)]
    wsem = [refs[_NRING * (2 + _BATCH) + g * _BATCH:]
            [:_BATCH] for g in range(_NRING)]
    gsem = refs[_NRING * (2 + _BATCH) + _NRING * _BATCH:][:_NRING]
    isem = refs[-1]

    wid = lax.axis_index("s") * _NC + lax.axis_index("c")
    t_base = wid * _T_PER_W

    idx_cps = [
        pltpu.async_copy(x_hbm.at[b, pl.ds(t_base, _T_PER_W)],
                         idx_v.at[b], isem)
        for b in range(_BATCH)
    ]
    for cp in idx_cps:
        cp.wait()

    def fire_chunk(ct):
        g = ct % _NRING
        t0 = t_base + ct * _TCH
        pos_cp = pltpu.async_copy(pos_hbm.at[pl.ds(t0, _TCH)], pos_v[g],
                                  psem[g])
        row_cps = []
        for b in range(_BATCH):
            idx = idx_v.at[b, pl.ds(ct * _TCH, _TCH)]
            row_cps.append(
                pltpu.async_copy(table_hbm.at[idx], rows_v[g][b], gsem[g]))
        return pos_cp, row_cps

    chunks = {k: fire_chunk(k) for k in range(_PREF)}
    writes = {}
    for ct in range(_NCH):
        g = ct % _NRING
        if ct + _PREF < _NCH:
            if ct + _PREF - _NRING in writes:
                for w in writes[ct + _PREF - _NRING]:
                    w.wait()
            chunks[ct + _PREF] = fire_chunk(ct + _PREF)
        pos_cp, row_cps = chunks[ct]
        pos_cp.wait()
        for cp in row_cps:
            cp.wait()

        pv = pos_v[g]
        bufs = rows_v[g]

        @plsc.parallel_loop(0, _TCH * _COLS, unroll=4)
        def _slice(i):
            r = i // _COLS
            col = i - r * _COLS
            sl = pl.ds(col * _LANES, _LANES)
            p = pv[r, sl]
            for b in range(_BATCH):
                rb = bufs[b]
                rb[r, sl] = rb[r, sl] * _SCALE + p

        t0 = t_base + ct * _TCH
        writes[ct] = [
            pltpu.async_copy(bufs[b], out_hbm.at[b, pl.ds(t0, _TCH)],
                             wsem[g][b])
            for b in range(_BATCH)
        ]
    for ct in range(_NCH - _NRING, _NCH):
        for w in writes[ct]:
            w.wait()


def kernel(x, table):
    return _embed(x.astype(jnp.int32), table, jnp.asarray(_POS_ENC))
